# baseline (speedup 1.0000x reference)
import functools

import jax
import jax.numpy as jnp
from jax import lax
from jax.experimental import pallas as pl
from jax.experimental.pallas import tpu as pltpu
from jax.experimental.pallas import tpu_sc as plsc

L = 16


def _make_phase1(n_pix, n_chan, n_seg, hw, p_per_w, ka):
    mesh = plsc.VectorSubcoreMesh(core_axis_name="c", subcore_axis_name="s")
    n_chunks = p_per_w // ka
    vseg = n_seg // 16

    nacc = (n_chan + 1) * n_seg
    red = nacc // 16

    @functools.partial(
        pl.kernel,
        out_type=jax.ShapeDtypeStruct((2, nacc), jnp.float32),
        mesh=mesh,
        compiler_params=pltpu.CompilerParams(needs_layout_passes=False),
        scratch_types=[
            pltpu.VMEM((nacc,), jnp.float32),
            pltpu.VMEM((ka,), jnp.int32),
            pltpu.VMEM((ka // 512, 512), jnp.float32),
            pltpu.VMEM((ka // 512, 512), jnp.float32),
            pltpu.VMEM((ka // 512, 512), jnp.float32),
            pltpu.VMEM((16, red), jnp.float32),
            pltpu.VMEM((red,), jnp.float32),
            pltpu.VMEM_SHARED((16, nacc), jnp.float32),
        ],
    )
    def phase1(img_hbm, seg_hbm, part_out, acc, segc, xc0, xc1, xc2,
               rbuf, obuf, acc_sh):
        xc = [xc0, xc1, xc2]
        cid = lax.axis_index("c")
        sid = lax.axis_index("s")
        wid = cid * 16 + sid
        b = wid // (hw // p_per_w)
        base = wid * p_per_w

        zf = jnp.zeros((L,), jnp.float32)

        def zbody(j, _):
            acc[pl.ds(j * L, L)] = zf
            return ()
        lax.fori_loop(0, (n_chan + 1) * n_seg // L, zbody, ())

        ones = jnp.ones((L,), jnp.float32)
        off_c = [jnp.full((L,), cc * n_seg, jnp.int32)
                 for cc in range(n_chan + 1)]

        def flush(cur, sums, nv):
            for cc in range(n_chan):
                plsc.addupdate_scatter(acc, [cur + off_c[cc]], sums[cc])
            plsc.addupdate_scatter(acc, [cur + off_c[n_chan]], nv)

        zf32 = jnp.zeros((L,), jnp.float32)
        last_idx = jnp.full((L,), L - 1, jnp.int32)

        carry0 = (jnp.zeros((L,), jnp.int32),) + tuple(
            zf32 for _ in range(n_chan + 1))

        def chunk_body(ch, carry):
            off = ch * ka
            pltpu.sync_copy(seg_hbm.at[pl.ds(base + off, ka)], segc)
            row0 = pl.multiple_of((base - b * hw + off) // 512, 8)
            for cc in range(n_chan):
                pltpu.sync_copy(
                    img_hbm.at[b * n_chan + cc,
                               pl.ds(row0, ka // 512), :],
                    xc[cc])

            def row_body(rw, carry):
                for u in range(512 // L):
                    cur = carry[0]
                    sv = segc[pl.ds(rw * 512 + u * L, L)]
                    xs = [xc[cc][rw, pl.ds(u * L, L)]
                          for cc in range(n_chan)]
                    bnd = jnp.logical_not(jnp.all(sv == cur))

                    @pl.when(bnd)
                    def _():
                        flush(cur, carry[1:n_chan + 1], carry[n_chan + 1])
                        for cc in range(n_chan):
                            plsc.addupdate_scatter(
                                acc, [sv + off_c[cc]], xs[cc])
                        plsc.addupdate_scatter(
                            acc, [sv + off_c[n_chan]], ones)

                    mv = jnp.broadcast_to(bnd, (L,))
                    new_cur = sv.at[last_idx].get(
                        mode="promise_in_bounds")
                    carry = (
                        (jnp.where(mv, new_cur, cur),)
                        + tuple(jnp.where(mv, zf32, s + x)
                                for s, x in zip(carry[1:], xs + [ones])))
                return carry
            return lax.fori_loop(0, ka // 512, row_body, carry)
        carry = lax.fori_loop(0, n_chunks, chunk_body, carry0)
        flush(carry[0], carry[1:n_chan + 1], carry[n_chan + 1])

        pltpu.sync_copy(acc, acc_sh.at[sid])
        plsc.subcore_barrier()
        pltpu.sync_copy(acc_sh.at[:, pl.ds(sid * red, red)], rbuf)

        def rbody(k, _):
            sl = pl.ds(k * L, L)
            v = rbuf[0, sl]
            for t in range(1, 16):
                v = v + rbuf[t, sl]
            obuf[sl] = v
            return ()
        lax.fori_loop(0, red // L, rbody, ())
        pltpu.sync_copy(obuf, part_out.at[cid, pl.ds(sid * red, red)])

    return phase1


def _make_phase2(n_pix, n_chan, n_seg, hw, p_per_w, kb):
    mesh = plsc.VectorSubcoreMesh(core_axis_name="c", subcore_axis_name="s")
    n_chunks = p_per_w // kb
    vseg = n_seg // 16

    @functools.partial(
        pl.kernel,
        out_type=jax.ShapeDtypeStruct((n_pix * 4,), jnp.float32),
        mesh=mesh,
        compiler_params=pltpu.CompilerParams(needs_layout_passes=False),
        scratch_types=[
            pltpu.VMEM((2, n_chan + 1, vseg), jnp.float32),
            pltpu.VMEM((vseg * n_chan,), jnp.float32),
            pltpu.VMEM((n_chan * vseg,), jnp.float32),
            pltpu.VMEM((n_chan * n_seg,), jnp.float32),
            pltpu.VMEM((kb,), jnp.int32),
            pltpu.VMEM((kb // 512, 512), jnp.float32),
            pltpu.VMEM((kb // 512, 512), jnp.float32),
            pltpu.VMEM((kb // 512, 512), jnp.float32),
            pltpu.VMEM((kb * 4,), jnp.float32),
            pltpu.VMEM_SHARED((n_chan * n_seg,), jnp.float32),
        ],
    )
    def phase2(part_hbm, fvr_hbm, img_hbm, seg_hbm, out_hbm,
               pbuf, fbuf, rsl, rvm, segc, xc0, xc1, xc2, oc, r_sh):
        xc = [xc0, xc1, xc2]
        cid = lax.axis_index("c")
        sid = lax.axis_index("s")
        wid = cid * 16 + sid
        b = wid // (hw // p_per_w)
        base = wid * p_per_w

        pltpu.sync_copy(part_hbm.at[:, :, pl.ds(sid * vseg, vseg)], pbuf)
        pltpu.sync_copy(fvr_hbm.at[pl.ds(sid * vseg * n_chan,
                                         vseg * n_chan)], fbuf)
        iota = lax.iota(jnp.int32, L)
        onef = jnp.ones((L,), jnp.float32)

        def rbody(k, _):
            sl = pl.ds(k * L, L)
            cnt = pbuf[0, n_chan, sl] + pbuf[1, n_chan, sl]
            inv = onef / jnp.maximum(cnt, onef)
            vpos = (iota + k * L) * n_chan
            for cc in range(n_chan):
                ssum = pbuf[0, cc, sl] + pbuf[1, cc, sl]
                fv = plsc.load_gather(fbuf, [vpos + cc])
                rsl[pl.ds(cc * vseg + k * L, L)] = fv - ssum * inv
            return ()
        lax.fori_loop(0, vseg // L, rbody, ())
        for cc in range(n_chan):
            pltpu.sync_copy(
                rsl.at[pl.ds(cc * vseg, vseg)],
                r_sh.at[pl.ds(cc * n_seg + sid * vseg, vseg)])
        plsc.subcore_barrier()
        pltpu.sync_copy(r_sh, rvm)

        off_c = [jnp.full((L,), cc * n_seg, jnp.int32) for cc in range(n_chan)]
        zf32 = jnp.zeros((L,), jnp.float32)
        last_idx = jnp.full((L,), L - 1, jnp.int32)
        carry0 = (jnp.full((L,), -1, jnp.int32),) + tuple(
            zf32 for _ in range(n_chan))

        def chunk_body(ch, carry):
            off = ch * kb
            pltpu.sync_copy(seg_hbm.at[pl.ds(base + off, kb)], segc)
            row0 = pl.multiple_of((base - b * hw + off) // 512, 8)
            for cc in range(n_chan):
                pltpu.sync_copy(
                    img_hbm.at[b * n_chan + cc,
                               pl.ds(row0, kb // 512), :],
                    xc[cc])

            def row_body(rw, carry):
                for u in range(512 // L):
                    cur = carry[0]
                    sv = segc[pl.ds(rw * 512 + u * L, L)]
                    bnd = jnp.logical_not(jnp.all(sv == cur))
                    mv = jnp.broadcast_to(bnd, (L,))
                    tp = rw * 2048 + (u // 8) * 512 + (u % 8) * L
                    new_cur = sv.at[last_idx].get(mode="promise_in_bounds")
                    ncarry = [jnp.where(mv, new_cur, cur)]
                    for cc in range(n_chan):
                        xv = xc[cc][rw, pl.ds(u * L, L)]
                        rv = plsc.load_gather(rvm, [sv + off_c[cc]], mask=mv)
                        rs = jnp.where(mv, rv, carry[1 + cc])
                        oc[pl.ds(tp + cc * 128, L)] = xv + rs
                        rn = rv.at[last_idx].get(mode="promise_in_bounds")
                        ncarry.append(jnp.where(mv, rn, carry[1 + cc]))
                    carry = tuple(ncarry)
                return carry
            carry = lax.fori_loop(0, kb // 512, row_body, carry)
            pltpu.sync_copy(oc, out_hbm.at[pl.ds((base + off) * 4, kb * 4)])
            return carry
        lax.fori_loop(0, n_chunks, chunk_body, carry0)

    return phase2


@jax.jit
def kernel(img, fV_region, seg):
    B, C, H, W = img.shape
    nV = fV_region.shape[0]
    N = B * H * W
    HW = H * W
    NW = 32
    P = N // NW

    img2 = img.reshape(B * C, H, W)
    fvr = fV_region.reshape(-1)

    part = _make_phase1(N, C, nV, HW, P, 4096)(img2, seg)
    part3 = part.reshape(2, C + 1, nV)
    out = _make_phase2(N, C, nV, HW, P, 4096)(part3, fvr, img2, seg)
    o4 = out.reshape(N // 128, 4, 128)
    return o4[:, :C, :].transpose(0, 2, 1).reshape(N, C)


# speedup vs baseline: 1.1956x; 1.1956x over previous
import functools

import jax
import jax.numpy as jnp
from jax import lax
from jax.experimental import pallas as pl
from jax.experimental.pallas import tpu as pltpu
from jax.experimental.pallas import tpu_sc as plsc

L = 16


def _make_phase1(n_pix, n_chan, n_seg, hw, p_per_w, ka):
    mesh = plsc.VectorSubcoreMesh(core_axis_name="c", subcore_axis_name="s")
    n_chunks = p_per_w // ka
    vseg = n_seg // 16

    nacc = (n_chan + 1) * n_seg
    red = nacc // 16

    @functools.partial(
        pl.kernel,
        out_type=jax.ShapeDtypeStruct((2, nacc), jnp.float32),
        mesh=mesh,
        compiler_params=pltpu.CompilerParams(needs_layout_passes=False),
        scratch_types=[
            pltpu.VMEM((nacc,), jnp.float32),
            pltpu.VMEM((ka,), jnp.int32),
            pltpu.VMEM((ka // 512, 512), jnp.float32),
            pltpu.VMEM((ka // 512, 512), jnp.float32),
            pltpu.VMEM((ka // 512, 512), jnp.float32),
            pltpu.VMEM((16, red), jnp.float32),
            pltpu.VMEM((red,), jnp.float32),
            pltpu.VMEM_SHARED((16, nacc), jnp.float32),
        ],
    )
    def phase1(img_hbm, seg_hbm, part_out, acc, segc, xc0, xc1, xc2,
               rbuf, obuf, acc_sh):
        xc = [xc0, xc1, xc2]
        cid = lax.axis_index("c")
        sid = lax.axis_index("s")
        wid = cid * 16 + sid
        b = wid // (hw // p_per_w)
        base = wid * p_per_w

        zf = jnp.zeros((L,), jnp.float32)

        def zbody(j, _):
            acc[pl.ds(j * L, L)] = zf
            return ()
        lax.fori_loop(0, (n_chan + 1) * n_seg // L, zbody, ())

        ones = jnp.ones((L,), jnp.float32)
        off_c = [jnp.full((L,), cc * n_seg, jnp.int32)
                 for cc in range(n_chan + 1)]

        def flush(cur, sums, nv):
            for cc in range(n_chan):
                plsc.addupdate_scatter(acc, [cur + off_c[cc]], sums[cc])
            plsc.addupdate_scatter(acc, [cur + off_c[n_chan]], nv)

        zf32 = jnp.zeros((L,), jnp.float32)
        last_idx = jnp.full((L,), L - 1, jnp.int32)

        carry0 = (jnp.zeros((L,), jnp.int32),) + tuple(
            zf32 for _ in range(n_chan + 1))

        def chunk_body(ch, carry):
            off = ch * ka
            pltpu.sync_copy(seg_hbm.at[pl.ds(base + off, ka)], segc)
            row0 = pl.multiple_of((base - b * hw + off) // 512, 8)
            for cc in range(n_chan):
                pltpu.sync_copy(
                    img_hbm.at[b * n_chan + cc,
                               pl.ds(row0, ka // 512), :],
                    xc[cc])

            def row_body(rw, carry):
                for u in range(512 // L):
                    cur = carry[0]
                    sv = segc[pl.ds(rw * 512 + u * L, L)]
                    xs = [xc[cc][rw, pl.ds(u * L, L)]
                          for cc in range(n_chan)]
                    bnd = jnp.logical_not(jnp.all(sv == cur))

                    @pl.when(bnd)
                    def _():
                        flush(cur, carry[1:n_chan + 1], carry[n_chan + 1])
                        for cc in range(n_chan):
                            plsc.addupdate_scatter(
                                acc, [sv + off_c[cc]], xs[cc])
                        plsc.addupdate_scatter(
                            acc, [sv + off_c[n_chan]], ones)

                    mv = jnp.broadcast_to(bnd, (L,))
                    new_cur = sv.at[last_idx].get(
                        mode="promise_in_bounds")
                    carry = (
                        (jnp.where(mv, new_cur, cur),)
                        + tuple(jnp.where(mv, zf32, s + x)
                                for s, x in zip(carry[1:], xs + [ones])))
                return carry
            return lax.fori_loop(0, ka // 512, row_body, carry)
        carry = lax.fori_loop(0, n_chunks, chunk_body, carry0)
        flush(carry[0], carry[1:n_chan + 1], carry[n_chan + 1])

        pltpu.sync_copy(acc, acc_sh.at[sid])
        plsc.subcore_barrier()
        pltpu.sync_copy(acc_sh.at[:, pl.ds(sid * red, red)], rbuf)

        def rbody(k, _):
            sl = pl.ds(k * L, L)
            v = rbuf[0, sl]
            for t in range(1, 16):
                v = v + rbuf[t, sl]
            obuf[sl] = v
            return ()
        lax.fori_loop(0, red // L, rbody, ())
        pltpu.sync_copy(obuf, part_out.at[cid, pl.ds(sid * red, red)])

    return phase1


def _make_phase2(n_pix, n_chan, n_seg, hw, p_per_w, kb):
    mesh = plsc.VectorSubcoreMesh(core_axis_name="c", subcore_axis_name="s")
    n_chunks = p_per_w // kb
    vseg = n_seg // 16

    @functools.partial(
        pl.kernel,
        out_type=jax.ShapeDtypeStruct((n_pix * 4,), jnp.float32),
        mesh=mesh,
        compiler_params=pltpu.CompilerParams(needs_layout_passes=False),
        scratch_types=[
            pltpu.VMEM((2, n_chan + 1, vseg), jnp.float32),
            pltpu.VMEM((vseg * n_chan,), jnp.float32),
            pltpu.VMEM((n_chan * vseg,), jnp.float32),
            pltpu.VMEM((n_chan * n_seg,), jnp.float32),
            pltpu.VMEM((kb,), jnp.int32),
            pltpu.VMEM((kb // 512, 512), jnp.float32),
            pltpu.VMEM((kb // 512, 512), jnp.float32),
            pltpu.VMEM((kb // 512, 512), jnp.float32),
            pltpu.VMEM((kb * 4,), jnp.float32),
            pltpu.VMEM_SHARED((n_chan * n_seg,), jnp.float32),
        ],
    )
    def phase2(part_hbm, fvr_hbm, img_hbm, seg_hbm, out_hbm,
               pbuf, fbuf, rsl, rvm, segc, xc0, xc1, xc2, oc, r_sh):
        xc = [xc0, xc1, xc2]
        cid = lax.axis_index("c")
        sid = lax.axis_index("s")
        wid = cid * 16 + sid
        b = wid // (hw // p_per_w)
        base = wid * p_per_w

        pltpu.sync_copy(part_hbm.at[:, :, pl.ds(sid * vseg, vseg)], pbuf)
        pltpu.sync_copy(fvr_hbm.at[pl.ds(sid * vseg * n_chan,
                                         vseg * n_chan)], fbuf)
        iota = lax.iota(jnp.int32, L)
        onef = jnp.ones((L,), jnp.float32)

        def rbody(k, _):
            sl = pl.ds(k * L, L)
            cnt = pbuf[0, n_chan, sl] + pbuf[1, n_chan, sl]
            inv = onef / jnp.maximum(cnt, onef)
            vpos = (iota + k * L) * n_chan
            for cc in range(n_chan):
                ssum = pbuf[0, cc, sl] + pbuf[1, cc, sl]
                fv = plsc.load_gather(fbuf, [vpos + cc])
                rsl[pl.ds(cc * vseg + k * L, L)] = fv - ssum * inv
            return ()
        lax.fori_loop(0, vseg // L, rbody, ())
        for cc in range(n_chan):
            pltpu.sync_copy(
                rsl.at[pl.ds(cc * vseg, vseg)],
                r_sh.at[pl.ds(cc * n_seg + sid * vseg, vseg)])
        plsc.subcore_barrier()
        pltpu.sync_copy(r_sh, rvm)

        off_c = [jnp.full((L,), cc * n_seg, jnp.int32) for cc in range(n_chan)]

        def chunk_body(ch, _):
            off = ch * kb
            pltpu.sync_copy(seg_hbm.at[pl.ds(base + off, kb)], segc)
            row0 = pl.multiple_of((base - b * hw + off) // 512, 8)
            for cc in range(n_chan):
                pltpu.sync_copy(
                    img_hbm.at[b * n_chan + cc,
                               pl.ds(row0, kb // 512), :],
                    xc[cc])

            def row_body(rw, _):
                for u in range(512 // L):
                    sv = segc[pl.ds(rw * 512 + u * L, L)]
                    tp = rw * 2048 + (u // 8) * 512 + (u % 8) * L
                    for cc in range(n_chan):
                        xv = xc[cc][rw, pl.ds(u * L, L)]
                        rv = plsc.load_gather(rvm, [sv + off_c[cc]])
                        oc[pl.ds(tp + cc * 128, L)] = xv + rv
                return ()
            lax.fori_loop(0, kb // 512, row_body, ())
            pltpu.sync_copy(oc, out_hbm.at[pl.ds((base + off) * 4, kb * 4)])
            return ()
        lax.fori_loop(0, n_chunks, chunk_body, ())

    return phase2


@jax.jit
def kernel(img, fV_region, seg):
    B, C, H, W = img.shape
    nV = fV_region.shape[0]
    N = B * H * W
    HW = H * W
    NW = 32
    P = N // NW

    img2 = img.reshape(B * C, H, W)
    fvr = fV_region.reshape(-1)

    part = _make_phase1(N, C, nV, HW, P, 4096)(img2, seg)
    part3 = part.reshape(2, C + 1, nV)
    out = _make_phase2(N, C, nV, HW, P, 4096)(part3, fvr, img2, seg)
    o4 = out.reshape(N // 128, 4, 128)
    return o4[:, :C, :].transpose(0, 2, 1).reshape(N, C)


# speedup vs baseline: 1.6156x; 1.3513x over previous
import functools

import jax
import jax.numpy as jnp
from jax import lax
from jax.experimental import pallas as pl
from jax.experimental.pallas import tpu as pltpu
from jax.experimental.pallas import tpu_sc as plsc

L = 16


def _make_phase1(n_pix, n_chan, n_seg, hw, p_per_w, ka):
    mesh = plsc.VectorSubcoreMesh(core_axis_name="c", subcore_axis_name="s")
    n_chunks = p_per_w // ka
    vseg = n_seg // 16

    nacc = (n_chan + 1) * n_seg
    red = nacc // 16

    @functools.partial(
        pl.kernel,
        out_type=jax.ShapeDtypeStruct((2, nacc), jnp.float32),
        mesh=mesh,
        compiler_params=pltpu.CompilerParams(needs_layout_passes=False),
        scratch_types=[
            pltpu.VMEM((nacc,), jnp.float32),
            pltpu.VMEM((ka,), jnp.int32),
            pltpu.VMEM((ka // 512, 512), jnp.float32),
            pltpu.VMEM((ka // 512, 512), jnp.float32),
            pltpu.VMEM((ka // 512, 512), jnp.float32),
            pltpu.VMEM((ka,), jnp.int32),
            pltpu.VMEM((ka // 512, 512), jnp.float32),
            pltpu.VMEM((ka // 512, 512), jnp.float32),
            pltpu.VMEM((ka // 512, 512), jnp.float32),
            pltpu.VMEM((16, 512), jnp.float32),
            pltpu.VMEM((red,), jnp.float32),
            pltpu.VMEM_SHARED((16, nacc), jnp.float32),
            pltpu.SemaphoreType.DMA,
            pltpu.SemaphoreType.DMA,
        ],
    )
    def phase1(img_hbm, seg_hbm, part_out, acc,
               segcA, xa0, xa1, xa2, segcB, xb0, xb1, xb2,
               rbuf, obuf, acc_sh, semA, semB):
        slots = [(segcA, [xa0, xa1, xa2], semA),
                 (segcB, [xb0, xb1, xb2], semB)]
        cid = lax.axis_index("c")
        sid = lax.axis_index("s")
        wid = cid * 16 + sid
        b = wid // (hw // p_per_w)
        base = wid * p_per_w

        zf = jnp.zeros((L,), jnp.float32)

        def zbody(j, _):
            acc[pl.ds(j * L, L)] = zf
            return ()
        lax.fori_loop(0, (n_chan + 1) * n_seg // L, zbody, ())

        ones = jnp.ones((L,), jnp.float32)
        off_c = [jnp.full((L,), cc * n_seg, jnp.int32)
                 for cc in range(n_chan + 1)]

        def flush(cur, sums, nv):
            for cc in range(n_chan):
                plsc.addupdate_scatter(acc, [cur + off_c[cc]], sums[cc])
            plsc.addupdate_scatter(acc, [cur + off_c[n_chan]], nv)

        zf32 = jnp.zeros((L,), jnp.float32)
        last_idx = jnp.full((L,), L - 1, jnp.int32)

        carry0 = (jnp.zeros((L,), jnp.int32),) + tuple(
            zf32 for _ in range(n_chan + 1))

        def issue(ch, slot):
            segc, xc, sem = slots[slot]
            off = ch * ka
            row0 = pl.multiple_of((base - b * hw + off) // 512, 8)
            pltpu.async_copy(seg_hbm.at[pl.ds(base + off, ka)], segc, sem)
            for cc in range(n_chan):
                pltpu.async_copy(
                    img_hbm.at[b * n_chan + cc,
                               pl.ds(row0, ka // 512), :],
                    xc[cc], sem)

        def drain(ch, slot):
            segc, xc, sem = slots[slot]
            off = ch * ka
            row0 = pl.multiple_of((base - b * hw + off) // 512, 8)
            pltpu.make_async_copy(
                seg_hbm.at[pl.ds(base + off, ka)], segc, sem).wait()
            for cc in range(n_chan):
                pltpu.make_async_copy(
                    img_hbm.at[b * n_chan + cc,
                               pl.ds(row0, ka // 512), :],
                    xc[cc], sem).wait()

        def compute(slot, carry):
            segc, xc, _ = slots[slot]

            def row_body(rw, carry):
                for u in range(512 // L):
                    cur = carry[0]
                    sv = segc[pl.ds(rw * 512 + u * L, L)]
                    xs = [xc[cc][rw, pl.ds(u * L, L)]
                          for cc in range(n_chan)]
                    bnd = jnp.logical_not(jnp.all(sv == cur))

                    @pl.when(bnd)
                    def _():
                        flush(cur, carry[1:n_chan + 1], carry[n_chan + 1])
                        for cc in range(n_chan):
                            plsc.addupdate_scatter(
                                acc, [sv + off_c[cc]], xs[cc])
                        plsc.addupdate_scatter(
                            acc, [sv + off_c[n_chan]], ones)

                    mv = jnp.broadcast_to(bnd, (L,))
                    new_cur = sv.at[last_idx].get(
                        mode="promise_in_bounds")
                    carry = (
                        (jnp.where(mv, new_cur, cur),)
                        + tuple(jnp.where(mv, zf32, s + x)
                                for s, x in zip(carry[1:], xs + [ones])))
                return carry
            return lax.fori_loop(0, ka // 512, row_body, carry)

        issue(0, 0)

        def pair_body(p, carry):
            a = p * 2
            issue(a + 1, 1)
            drain(a, 0)
            carry = compute(0, carry)

            @pl.when(a + 2 < n_chunks)
            def _():
                issue(a + 2, 0)
            drain(a + 1, 1)
            carry = compute(1, carry)
            return carry
        carry = lax.fori_loop(0, n_chunks // 2, pair_body, carry0)
        flush(carry[0], carry[1:n_chan + 1], carry[n_chan + 1])

        pltpu.sync_copy(acc, acc_sh.at[sid])
        plsc.subcore_barrier()
        for p in range(red // 512):
            pltpu.sync_copy(
                acc_sh.at[:, pl.ds(sid * red + p * 512, 512)], rbuf)

            def rbody(k, _, p=p):
                sl = pl.ds(k * L, L)
                v = rbuf[0, sl]
                for t in range(1, 16):
                    v = v + rbuf[t, sl]
                obuf[pl.ds(p * 512 + k * L, L)] = v
                return ()
            lax.fori_loop(0, 512 // L, rbody, ())
        pltpu.sync_copy(obuf, part_out.at[cid, pl.ds(sid * red, red)])

    return phase1


def _make_phase2(n_pix, n_chan, n_seg, hw, p_per_w, kb):
    mesh = plsc.VectorSubcoreMesh(core_axis_name="c", subcore_axis_name="s")
    n_chunks = p_per_w // kb
    vseg = n_seg // 16

    @functools.partial(
        pl.kernel,
        out_type=jax.ShapeDtypeStruct((n_pix * 4,), jnp.float32),
        mesh=mesh,
        compiler_params=pltpu.CompilerParams(needs_layout_passes=False),
        scratch_types=[
            pltpu.VMEM((2, n_chan + 1, vseg), jnp.float32),
            pltpu.VMEM((vseg * n_chan,), jnp.float32),
            pltpu.VMEM((n_chan * vseg,), jnp.float32),
            pltpu.VMEM((n_chan * n_seg,), jnp.float32),
            pltpu.VMEM((kb,), jnp.int32),
            pltpu.VMEM((kb // 512, 512), jnp.float32),
            pltpu.VMEM((kb // 512, 512), jnp.float32),
            pltpu.VMEM((kb // 512, 512), jnp.float32),
            pltpu.VMEM((kb,), jnp.int32),
            pltpu.VMEM((kb // 512, 512), jnp.float32),
            pltpu.VMEM((kb // 512, 512), jnp.float32),
            pltpu.VMEM((kb // 512, 512), jnp.float32),
            pltpu.VMEM((kb * 4,), jnp.float32),
            pltpu.VMEM((kb * 4,), jnp.float32),
            pltpu.VMEM_SHARED((n_chan * n_seg,), jnp.float32),
            pltpu.SemaphoreType.DMA,
            pltpu.SemaphoreType.DMA,
            pltpu.SemaphoreType.DMA,
            pltpu.SemaphoreType.DMA,
        ],
    )
    def phase2(part_hbm, fvr_hbm, img_hbm, seg_hbm, out_hbm,
               pbuf, fbuf, rsl, rvm,
               segcA, xa0, xa1, xa2, segcB, xb0, xb1, xb2,
               ocA, ocB, r_sh, semA, semB, semOA, semOB):
        slots = [(segcA, [xa0, xa1, xa2], semA),
                 (segcB, [xb0, xb1, xb2], semB)]
        ocs = [(ocA, semOA), (ocB, semOB)]
        cid = lax.axis_index("c")
        sid = lax.axis_index("s")
        wid = cid * 16 + sid
        b = wid // (hw // p_per_w)
        base = wid * p_per_w

        pltpu.sync_copy(part_hbm.at[:, :, pl.ds(sid * vseg, vseg)], pbuf)
        pltpu.sync_copy(fvr_hbm.at[pl.ds(sid * vseg * n_chan,
                                         vseg * n_chan)], fbuf)
        iota = lax.iota(jnp.int32, L)
        onef = jnp.ones((L,), jnp.float32)

        def rbody(k, _):
            sl = pl.ds(k * L, L)
            cnt = pbuf[0, n_chan, sl] + pbuf[1, n_chan, sl]
            inv = onef / jnp.maximum(cnt, onef)
            vpos = (iota + k * L) * n_chan
            for cc in range(n_chan):
                ssum = pbuf[0, cc, sl] + pbuf[1, cc, sl]
                fv = plsc.load_gather(fbuf, [vpos + cc])
                rsl[pl.ds(cc * vseg + k * L, L)] = fv - ssum * inv
            return ()
        lax.fori_loop(0, vseg // L, rbody, ())
        for cc in range(n_chan):
            pltpu.sync_copy(
                rsl.at[pl.ds(cc * vseg, vseg)],
                r_sh.at[pl.ds(cc * n_seg + sid * vseg, vseg)])
        plsc.subcore_barrier()
        pltpu.sync_copy(r_sh, rvm)

        off_c = [jnp.full((L,), cc * n_seg, jnp.int32) for cc in range(n_chan)]

        def issue(ch, slot):
            segc, xc, sem = slots[slot]
            off = ch * kb
            row0 = pl.multiple_of((base - b * hw + off) // 512, 8)
            pltpu.async_copy(seg_hbm.at[pl.ds(base + off, kb)], segc, sem)
            for cc in range(n_chan):
                pltpu.async_copy(
                    img_hbm.at[b * n_chan + cc,
                               pl.ds(row0, kb // 512), :],
                    xc[cc], sem)

        def drain(ch, slot):
            segc, xc, sem = slots[slot]
            off = ch * kb
            row0 = pl.multiple_of((base - b * hw + off) // 512, 8)
            pltpu.make_async_copy(
                seg_hbm.at[pl.ds(base + off, kb)], segc, sem).wait()
            for cc in range(n_chan):
                pltpu.make_async_copy(
                    img_hbm.at[b * n_chan + cc,
                               pl.ds(row0, kb // 512), :],
                    xc[cc], sem).wait()

        def out_dst(ch, slot):
            return (ocs[slot][0],
                    out_hbm.at[pl.ds((base + ch * kb) * 4, kb * 4)],
                    ocs[slot][1])

        def compute(ch, slot):
            segc, xc, _ = slots[slot]
            oc = ocs[slot][0]

            def row_body(rw, _):
                for u in range(512 // L):
                    sv = segc[pl.ds(rw * 512 + u * L, L)]
                    tp = rw * 2048 + (u // 8) * 512 + (u % 8) * L
                    for cc in range(n_chan):
                        xv = xc[cc][rw, pl.ds(u * L, L)]
                        rv = plsc.load_gather(rvm, [sv + off_c[cc]])
                        oc[pl.ds(tp + cc * 128, L)] = xv + rv
                return ()
            lax.fori_loop(0, kb // 512, row_body, ())

        issue(0, 0)

        def pair_body(p, _):
            a = p * 2
            issue(a + 1, 1)
            drain(a, 0)

            @pl.when(p > 0)
            def _():
                pltpu.make_async_copy(*out_dst(a - 2, 0)).wait()
            compute(a, 0)
            pltpu.async_copy(*out_dst(a, 0))

            @pl.when(a + 2 < n_chunks)
            def _():
                issue(a + 2, 0)
            drain(a + 1, 1)

            @pl.when(p > 0)
            def _():
                pltpu.make_async_copy(*out_dst(a - 1, 1)).wait()
            compute(a + 1, 1)
            pltpu.async_copy(*out_dst(a + 1, 1))
            return ()
        lax.fori_loop(0, n_chunks // 2, pair_body, ())
        pltpu.make_async_copy(*out_dst(n_chunks - 2, 0)).wait()
        pltpu.make_async_copy(*out_dst(n_chunks - 1, 1)).wait()

    return phase2


@jax.jit
def kernel(img, fV_region, seg):
    B, C, H, W = img.shape
    nV = fV_region.shape[0]
    N = B * H * W
    HW = H * W
    NW = 32
    P = N // NW

    img2 = img.reshape(B * C, H, W)
    fvr = fV_region.reshape(-1)

    part = _make_phase1(N, C, nV, HW, P, 4096)(img2, seg)
    part3 = part.reshape(2, C + 1, nV)
    out = _make_phase2(N, C, nV, HW, P, 4096)(part3, fvr, img2, seg)
    o4 = out.reshape(N // 128, 4, 128)
    return o4[:, :C, :].transpose(0, 2, 1).reshape(N, C)
